# Initial kernel scaffold; baseline (speedup 1.0000x reference)
import jax
import jax.numpy as jnp
from jax.experimental import pallas as pl


def kernel(node_features, edge_features, edge_index, params):
    raise NotImplementedError("write your pallas kernel here")


# speedup vs baseline: 2.9537x; 2.9537x over previous
import functools

import jax
import jax.numpy as jnp
from jax import lax
from jax.experimental import pallas as pl
from jax.experimental.pallas import tpu as pltpu
from jax.experimental.pallas import tpu_sc as plsc

N_NODES = 10000
N_PAD = 10240
N_EDGES = 320000
D = 128

NC, NS = 2, 16
NW = NC * NS
CHUNK = 80
N_CHUNKS = N_EDGES // CHUNK
CPW = N_CHUNKS // NW
CPS = N_CHUNKS // NS
ROWS_PER_SUB = N_PAD // NS

BN = 1280
BE = 1280


def _swish(x):
    return x * jax.nn.sigmoid(x)


def _ln(h, scale, off):
    mu = jnp.mean(h, axis=-1, keepdims=True)
    var = jnp.mean((h - mu) ** 2, axis=-1, keepdims=True)
    return (h - mu) * lax.rsqrt(var + 1e-5) * scale + off


def _dot(a, b):
    return jnp.dot(a, b, preferred_element_type=jnp.float32)


def _full(shape):
    return pl.BlockSpec(shape, lambda i: (0, 0))


def _rows(block):
    return pl.BlockSpec((block, D), lambda i: (i, 0))


def _mlp3_kernel(x_ref, w0, b0, w1, b1, w2, b2, sc, of, o_ref, *, use_ln):
    h = _swish(_dot(x_ref[...], w0[...]) + b0[...])
    h = _swish(_dot(h, w1[...]) + b1[...])
    h = _dot(h, w2[...]) + b2[...]
    o_ref[...] = _ln(h, sc[...], of[...]) if use_ln else h


def _mlp3(x, w0, b0, w1, b1, w2, b2, sc, of, use_ln, block):
    n = x.shape[0]
    return pl.pallas_call(
        functools.partial(_mlp3_kernel, use_ln=use_ln),
        out_shape=jax.ShapeDtypeStruct((n, D), jnp.float32),
        grid=(n // block,),
        in_specs=[_rows(block), _full(w0.shape), _full((1, D)),
                  _full(w1.shape), _full((1, D)), _full(w2.shape),
                  _full((1, D)), _full((1, D)), _full((1, D))],
        out_specs=_rows(block),
    )(x, w0, b0, w1, b1, w2, b2, sc, of)


def _edge_update_kernel(e_ref, gs_ref, gr_ref, w0, b0, w1, b1, w2, b2, sc, of,
                        o_ref):
    e = e_ref[...]
    h = _swish(_dot(e, w0[...]) + gs_ref[...] + gr_ref[...] + b0[...])
    h = _swish(_dot(h, w1[...]) + b1[...])
    h = _dot(h, w2[...]) + b2[...]
    o_ref[...] = e + _ln(h, sc[...], of[...])


def _edge_update(e, gs, gr, w0, b0, w1, b1, w2, b2, sc, of):
    return pl.pallas_call(
        _edge_update_kernel,
        out_shape=jax.ShapeDtypeStruct((N_EDGES, D), jnp.float32),
        grid=(N_EDGES // BE,),
        in_specs=[_rows(BE), _rows(BE), _rows(BE),
                  _full((D, D)), _full((1, D)), _full((D, D)), _full((1, D)),
                  _full((D, D)), _full((1, D)), _full((1, D)), _full((1, D))],
        out_specs=_rows(BE),
    )(e, gs, gr, w0, b0, w1, b1, w2, b2, sc, of)


def _node_update_kernel(n_ref, parts_refs, w0s, b0, w1, b1, w2, b2, sc, of,
                        o_ref):
    n = n_ref[...]
    h = _dot(n, w0s[0][...]) + b0[...]
    for p_ref, w in zip(parts_refs, w0s[1:]):
        h = h + _dot(p_ref[...], w[...])
    h = _swish(h)
    h = _swish(_dot(h, w1[...]) + b1[...])
    h = _dot(h, w2[...]) + b2[...]
    o_ref[...] = n + _ln(h, sc[...], of[...])


def _node_update(n, parts, w0s, b0, w1, b1, w2, b2, sc, of):
    k = len(parts)

    def body(*refs):
        n_ref = refs[0]
        parts_refs = refs[1:1 + k]
        w0_refs = refs[1 + k:2 + 2 * k]
        rest = refs[2 + 2 * k:]
        _node_update_kernel(n_ref, parts_refs, w0_refs, *rest)

    return pl.pallas_call(
        body,
        out_shape=jax.ShapeDtypeStruct((N_PAD, D), jnp.float32),
        grid=(N_PAD // BN,),
        in_specs=[_rows(BN)] + [_rows(BN)] * k + [_full((D, D))] * (k + 1) +
                 [_full((1, D)), _full((D, D)), _full((1, D)),
                  _full((D, D)), _full((1, D)), _full((1, D)), _full((1, D))],
        out_specs=_rows(BN),
    )(n, *parts, *w0s, b0, w1, b1, w2, b2, sc, of)


def _project_kernel(n_ref, ws, wr, ps_ref, pr_ref):
    n = n_ref[...]
    ps_ref[...] = _dot(n, ws[...])
    pr_ref[...] = _dot(n, wr[...])


def _project(n, ws, wr):
    out = jax.ShapeDtypeStruct((N_PAD, D), jnp.float32)
    return pl.pallas_call(
        _project_kernel,
        out_shape=(out, out),
        grid=(N_PAD // BN,),
        in_specs=[_rows(BN), _full((D, D)), _full((D, D))],
        out_specs=(_rows(BN), _rows(BN)),
    )(n, ws, wr)


def _wid():
    return lax.axis_index("s") * NC + lax.axis_index("c")


@functools.lru_cache(maxsize=1)
def _sc_kernels():
    mesh = plsc.VectorSubcoreMesh(core_axis_name="c", subcore_axis_name="s",
                                  num_cores=NC, num_subcores=NS)

    @functools.partial(
        pl.kernel,
        out_type=(jax.ShapeDtypeStruct((N_EDGES, D), jnp.float32),
                  jax.ShapeDtypeStruct((N_EDGES, D), jnp.float32)),
        mesh=mesh,
        scratch_types=[
            pltpu.VMEM((CPW, CHUNK), jnp.int32),
            pltpu.VMEM((CPW, CHUNK), jnp.int32),
            pltpu.VMEM((CHUNK, D), jnp.float32),
            pltpu.VMEM((CHUNK, D), jnp.float32),
            pltpu.SemaphoreType.DMA,
        ],
    )
    def sc_gather2(ps_hbm, pr_hbm, sidx_hbm, ridx_hbm, gs_hbm, gr_hbm,
                   sidx_v, ridx_v, rows_s, rows_r, sem):
        w = _wid()
        c0 = w * CPW
        pltpu.sync_copy(sidx_hbm.at[w], sidx_v)
        pltpu.sync_copy(ridx_hbm.at[w], ridx_v)

        def body(i, _):
            d1 = pltpu.async_copy(ps_hbm.at[sidx_v.at[i]], rows_s, sem)
            d2 = pltpu.async_copy(pr_hbm.at[ridx_v.at[i]], rows_r, sem)
            d1.wait()
            d2.wait()
            base = (c0 + i) * CHUNK
            pltpu.sync_copy(rows_s, gs_hbm.at[pl.ds(base, CHUNK)])
            pltpu.sync_copy(rows_r, gr_hbm.at[pl.ds(base, CHUNK)])
            return ()

        lax.fori_loop(0, CPW, body, (), unroll=False)

    @functools.partial(
        pl.kernel,
        out_type=(jax.ShapeDtypeStruct((N_PAD, D), jnp.float32),
                  jax.ShapeDtypeStruct((N_PAD, D), jnp.float32)),
        mesh=mesh,
        scratch_types=[
            pltpu.VMEM_SHARED((N_PAD, D), jnp.float32),
            pltpu.VMEM((CPS, CHUNK), jnp.int32),
            pltpu.VMEM((CHUNK, D), jnp.float32),
        ],
    )
    def sc_segsum2(msgs_hbm, ridx_hbm, sidx_hbm, zeros_hbm, recv_hbm,
                   sent_hbm, acc, idx_v, rows_v):
        c = lax.axis_index("c")
        s = lax.axis_index("s")
        r0 = s * ROWS_PER_SUB
        pltpu.sync_copy(zeros_hbm.at[pl.ds(r0, ROWS_PER_SUB)],
                        acc.at[pl.ds(r0, ROWS_PER_SUB)])

        @pl.when(c == 0)
        def _():
            pltpu.sync_copy(ridx_hbm.at[s], idx_v)

        @pl.when(c == 1)
        def _():
            pltpu.sync_copy(sidx_hbm.at[s], idx_v)

        plsc.subcore_barrier()

        def body(i, _):
            base = (s * CPS + i) * CHUNK
            pltpu.sync_copy(msgs_hbm.at[pl.ds(base, CHUNK)], rows_v)
            pltpu.sync_copy(rows_v, acc.at[idx_v.at[i]], add=True)
            return ()

        lax.fori_loop(0, CPS, body, (), unroll=False)

        plsc.subcore_barrier()

        @pl.when(c == 0)
        def _():
            pltpu.sync_copy(acc.at[pl.ds(r0, ROWS_PER_SUB)],
                            recv_hbm.at[pl.ds(r0, ROWS_PER_SUB)])

        @pl.when(c == 1)
        def _():
            pltpu.sync_copy(acc.at[pl.ds(r0, ROWS_PER_SUB)],
                            sent_hbm.at[pl.ds(r0, ROWS_PER_SUB)])

    return sc_gather2, sc_segsum2


def _mlp_args(p):
    ls = p["layers"]
    sc = p.get("ln_scale")
    of = p.get("ln_offset")
    r = lambda v: v.reshape(1, D) if v is not None else None
    return (ls[0]["w"], r(ls[0]["b"]), ls[1]["w"], r(ls[1]["b"]),
            ls[2]["w"], r(ls[2]["b"]), sc, of)


def kernel(node_features, edge_features, edge_index, params):
    senders = edge_index[0]
    receivers = edge_index[1]
    sidx_g = senders.reshape(NW, CPW, CHUNK)
    ridx_g = receivers.reshape(NW, CPW, CHUNK)
    sidx_s = senders.reshape(NS, CPS, CHUNK)
    ridx_s = receivers.reshape(NS, CPS, CHUNK)
    zeros = jnp.zeros((N_PAD, D), jnp.float32)
    one_row = jnp.ones((1, D), jnp.float32)

    nf = jnp.zeros((N_PAD, D), jnp.float32).at[:N_NODES, :27].set(node_features)
    ef = jnp.zeros((N_EDGES, D), jnp.float32).at[:, :4].set(edge_features)

    def enc(p, x, block):
        w0, b0, w1, b1, w2, b2, sc, of = _mlp_args(p)
        w0p = jnp.zeros((D, D), jnp.float32).at[:w0.shape[0]].set(w0)
        return _mlp3(x, w0p, b0, w1, b1, w2, b2, sc.reshape(1, D),
                     of.reshape(1, D), True, block)

    nodes = enc(params["enc_embed_node"], nf, BN)
    edges = enc(params["enc_embed_edge"], ef, BE)

    def gn_step(p, nodes, edges, include_sent):
        sc_gather2, sc_segsum2 = _sc_kernels()
        ew0, eb0, ew1, eb1, ew2, eb2, esc, eof = _mlp_args(p["edge"])
        ps, pr = _project(nodes, ew0[D:2 * D], ew0[2 * D:])
        gs, gr = sc_gather2(ps, pr, sidx_g, ridx_g)
        new_edges = _edge_update(edges, gs, gr, ew0[:D], eb0, ew1, eb1,
                                 ew2, eb2, esc.reshape(1, D), eof.reshape(1, D))
        recv, sent = sc_segsum2(new_edges, ridx_s, sidx_s, zeros)
        nw0, nb0, nw1, nb1, nw2, nb2, nsc, nof = _mlp_args(p["node"])
        if include_sent:
            parts = [recv, sent]
            w0s = [nw0[:D], nw0[D:2 * D], nw0[2 * D:]]
        else:
            parts = [recv]
            w0s = [nw0[:D], nw0[D:2 * D]]
        new_nodes = _node_update(nodes, parts, w0s, nb0, nw1, nb1, nw2, nb2,
                                 nsc.reshape(1, D), nof.reshape(1, D))
        return new_nodes, new_edges

    nodes, edges = gn_step(params["enc_gn"], nodes, edges, False)
    for i in range(8):
        nodes, edges = gn_step(params["proc_gn"][i], nodes, edges, True)
    nodes, edges = gn_step(params["dec_gn"], nodes, edges, False)

    dp = params["dec_out"]
    ls = dp["layers"]
    w2p = jnp.zeros((D, D), jnp.float32).at[:, :5].set(ls[2]["w"])
    b2p = jnp.zeros((1, D), jnp.float32).at[0, :5].set(ls[2]["b"])
    out = _mlp3(nodes, ls[0]["w"], ls[0]["b"].reshape(1, D),
                ls[1]["w"], ls[1]["b"].reshape(1, D), w2p, b2p,
                one_row, one_row, False, BN)
    return out[:N_NODES, :5]


# speedup vs baseline: 3.7902x; 1.2832x over previous
import functools

import jax
import jax.numpy as jnp
from jax import lax
from jax.experimental import pallas as pl
from jax.experimental.pallas import tpu as pltpu
from jax.experimental.pallas import tpu_sc as plsc

N_NODES = 10000
N_PAD = 10240
N_EDGES = 320000
D = 128

NC, NS = 2, 16
NW = NC * NS
CHUNK = 80
N_CHUNKS = N_EDGES // CHUNK
CPW = N_CHUNKS // NW
CPS = N_CHUNKS // NS
ROWS_PER_SUB = N_PAD // NS
NBUF = 5
NBUF_S = 2
N_GRP = 5
GRP_CH = CPS // N_GRP

BN = 1280
BE = 1280


def _swish(x):
    return x * jax.nn.sigmoid(x)


def _ln(h, scale, off):
    mu = jnp.mean(h, axis=-1, keepdims=True)
    var = jnp.mean((h - mu) ** 2, axis=-1, keepdims=True)
    return (h - mu) * lax.rsqrt(var + 1e-5) * scale + off


def _dot(a, b):
    return jnp.dot(a, b, preferred_element_type=jnp.float32)


def _full(shape):
    return pl.BlockSpec(shape, lambda i: (0, 0))


def _rows(block):
    return pl.BlockSpec((block, D), lambda i: (i, 0))


def _mlp3_kernel(x_ref, w0, b0, w1, b1, w2, b2, sc, of, o_ref, *, use_ln):
    h = _swish(_dot(x_ref[...], w0[...]) + b0[...])
    h = _swish(_dot(h, w1[...]) + b1[...])
    h = _dot(h, w2[...]) + b2[...]
    o_ref[...] = _ln(h, sc[...], of[...]) if use_ln else h


def _mlp3(x, w0, b0, w1, b1, w2, b2, sc, of, use_ln, block):
    n = x.shape[0]
    return pl.pallas_call(
        functools.partial(_mlp3_kernel, use_ln=use_ln),
        out_shape=jax.ShapeDtypeStruct((n, D), jnp.float32),
        grid=(n // block,),
        in_specs=[_rows(block), _full(w0.shape), _full((1, D)),
                  _full(w1.shape), _full((1, D)), _full(w2.shape),
                  _full((1, D)), _full((1, D)), _full((1, D))],
        out_specs=_rows(block),
    )(x, w0, b0, w1, b1, w2, b2, sc, of)


def _edge_update_kernel(e_ref, gs_ref, gr_ref, w0, b0, w1, b1, w2, b2, sc, of,
                        o_ref):
    e = e_ref[...]
    h = _swish(_dot(e, w0[...]) + gs_ref[...] + gr_ref[...] + b0[...])
    h = _swish(_dot(h, w1[...]) + b1[...])
    h = _dot(h, w2[...]) + b2[...]
    o_ref[...] = e + _ln(h, sc[...], of[...])


def _edge_update(e, gs, gr, w0, b0, w1, b1, w2, b2, sc, of):
    return pl.pallas_call(
        _edge_update_kernel,
        out_shape=jax.ShapeDtypeStruct((N_EDGES, D), jnp.float32),
        grid=(N_EDGES // BE,),
        in_specs=[_rows(BE), _rows(BE), _rows(BE),
                  _full((D, D)), _full((1, D)), _full((D, D)), _full((1, D)),
                  _full((D, D)), _full((1, D)), _full((1, D)), _full((1, D))],
        out_specs=_rows(BE),
    )(e, gs, gr, w0, b0, w1, b1, w2, b2, sc, of)


def _node_update_kernel(n_ref, parts_refs, w0s, b0, w1, b1, w2, b2, sc, of,
                        o_ref):
    n = n_ref[...]
    h = _dot(n, w0s[0][...]) + b0[...]
    for p_ref, w in zip(parts_refs, w0s[1:]):
        h = h + _dot(p_ref[...], w[...])
    h = _swish(h)
    h = _swish(_dot(h, w1[...]) + b1[...])
    h = _dot(h, w2[...]) + b2[...]
    o_ref[...] = n + _ln(h, sc[...], of[...])


def _node_update(n, parts, w0s, b0, w1, b1, w2, b2, sc, of):
    k = len(parts)

    def body(*refs):
        n_ref = refs[0]
        parts_refs = refs[1:1 + k]
        w0_refs = refs[1 + k:2 + 2 * k]
        rest = refs[2 + 2 * k:]
        _node_update_kernel(n_ref, parts_refs, w0_refs, *rest)

    return pl.pallas_call(
        body,
        out_shape=jax.ShapeDtypeStruct((N_PAD, D), jnp.float32),
        grid=(N_PAD // BN,),
        in_specs=[_rows(BN)] + [_rows(BN)] * k + [_full((D, D))] * (k + 1) +
                 [_full((1, D)), _full((D, D)), _full((1, D)),
                  _full((D, D)), _full((1, D)), _full((1, D)), _full((1, D))],
        out_specs=_rows(BN),
    )(n, *parts, *w0s, b0, w1, b1, w2, b2, sc, of)


def _project_kernel(n_ref, ws, wr, ps_ref, pr_ref):
    n = n_ref[...]
    ps_ref[...] = _dot(n, ws[...])
    pr_ref[...] = _dot(n, wr[...])


def _project(n, ws, wr):
    out = jax.ShapeDtypeStruct((N_PAD, D), jnp.float32)
    return pl.pallas_call(
        _project_kernel,
        out_shape=(out, out),
        grid=(N_PAD // BN,),
        in_specs=[_rows(BN), _full((D, D)), _full((D, D))],
        out_specs=(_rows(BN), _rows(BN)),
    )(n, ws, wr)


def _wid():
    return lax.axis_index("s") * NC + lax.axis_index("c")


@functools.lru_cache(maxsize=1)
def _sc_kernels():
    mesh = plsc.VectorSubcoreMesh(core_axis_name="c", subcore_axis_name="s",
                                  num_cores=NC, num_subcores=NS)

    @functools.partial(
        pl.kernel,
        out_type=(jax.ShapeDtypeStruct((N_EDGES, D), jnp.float32),
                  jax.ShapeDtypeStruct((N_EDGES, D), jnp.float32)),
        mesh=mesh,
        scratch_types=[
            pltpu.VMEM((CPS, CHUNK), jnp.int32),
            pltpu.VMEM((NBUF, CHUNK, D), jnp.float32),
            pltpu.SemaphoreType.DMA((NBUF,)),
            pltpu.SemaphoreType.DMA,
        ],
    )
    def sc_gather2(ps_hbm, pr_hbm, sidx_hbm, ridx_hbm, gs_hbm, gr_hbm,
                   idx_v, buf, sem_g, sem_w):
        c = lax.axis_index("c")
        s = lax.axis_index("s")

        def pipeline(tbl_hbm, idx3_hbm, out_hbm):
            pltpu.sync_copy(idx3_hbm.at[s], idx_v)
            for b in range(NBUF):
                pltpu.async_copy(tbl_hbm.at[idx_v.at[b]], buf.at[b],
                                 sem_g.at[b])

            def outer(g, _):
                for b in range(NBUF):
                    k = g * NBUF + b
                    pltpu.make_async_copy(
                        tbl_hbm.at[idx_v.at[k]], buf.at[b],
                        sem_g.at[b]).wait()
                    base = (s * CPS + k) * CHUNK
                    pltpu.async_copy(buf.at[b],
                                     out_hbm.at[pl.ds(base, CHUNK)],
                                     sem_w).wait()
                    nxt = k + NBUF

                    @pl.when(nxt < CPS)
                    def _():
                        pltpu.async_copy(tbl_hbm.at[idx_v.at[nxt]],
                                         buf.at[b], sem_g.at[b])
                return ()

            lax.fori_loop(0, CPS // NBUF, outer, (), unroll=False)

        @pl.when(c == 0)
        def _():
            pipeline(ps_hbm, sidx_hbm, gs_hbm)

        @pl.when(c == 1)
        def _():
            pipeline(pr_hbm, ridx_hbm, gr_hbm)

    @functools.partial(
        pl.kernel,
        out_type=(jax.ShapeDtypeStruct((N_PAD, D), jnp.float32),
                  jax.ShapeDtypeStruct((N_PAD, D), jnp.float32)),
        mesh=mesh,
        scratch_types=[
            pltpu.VMEM_SHARED((N_PAD, D), jnp.float32),
            pltpu.VMEM((GRP_CH, CHUNK), jnp.int32),
            pltpu.VMEM((NBUF_S, CHUNK, D), jnp.float32),
            pltpu.SemaphoreType.DMA((NBUF_S,)),
            pltpu.SemaphoreType.DMA,
        ],
    )
    def sc_segsum2(msgs_hbm, ridx_hbm, sidx_hbm, zeros_hbm, recv_hbm,
                   sent_hbm, acc, idx_v, rows, sem_g, sem_w):
        c = lax.axis_index("c")
        s = lax.axis_index("s")
        r0 = s * ROWS_PER_SUB
        pltpu.sync_copy(zeros_hbm.at[pl.ds(r0, ROWS_PER_SUB)],
                        acc.at[pl.ds(r0, ROWS_PER_SUB)])
        plsc.subcore_barrier()

        for grp in range(N_GRP):

            @pl.when(c == 0)
            def _():
                pltpu.sync_copy(ridx_hbm.at[s, grp], idx_v)

            @pl.when(c == 1)
            def _():
                pltpu.sync_copy(sidx_hbm.at[s, grp], idx_v)

            e0 = (s * CPS + grp * GRP_CH) * CHUNK
            for b in range(NBUF_S):
                pltpu.async_copy(msgs_hbm.at[pl.ds(e0 + b * CHUNK, CHUNK)],
                                 rows.at[b], sem_g.at[b])

            def outer(g, _):
                for b in range(NBUF_S):
                    k = g * NBUF_S + b
                    pltpu.make_async_copy(
                        msgs_hbm.at[pl.ds(0, CHUNK)], rows.at[b],
                        sem_g.at[b]).wait()
                    pltpu.async_copy(rows.at[b], acc.at[idx_v.at[k]], sem_w,
                                     add=True).wait()
                    nxt = k + NBUF_S

                    @pl.when(nxt < GRP_CH)
                    def _():
                        pltpu.async_copy(
                            msgs_hbm.at[pl.ds(e0 + nxt * CHUNK, CHUNK)],
                            rows.at[b], sem_g.at[b])
                return ()

            lax.fori_loop(0, GRP_CH // NBUF_S, outer, (), unroll=False)

        plsc.subcore_barrier()

        @pl.when(c == 0)
        def _():
            pltpu.sync_copy(acc.at[pl.ds(r0, ROWS_PER_SUB)],
                            recv_hbm.at[pl.ds(r0, ROWS_PER_SUB)])

        @pl.when(c == 1)
        def _():
            pltpu.sync_copy(acc.at[pl.ds(r0, ROWS_PER_SUB)],
                            sent_hbm.at[pl.ds(r0, ROWS_PER_SUB)])

    return sc_gather2, sc_segsum2


def _mlp_args(p):
    ls = p["layers"]
    sc = p.get("ln_scale")
    of = p.get("ln_offset")
    r = lambda v: v.reshape(1, D) if v is not None else None
    return (ls[0]["w"], r(ls[0]["b"]), ls[1]["w"], r(ls[1]["b"]),
            ls[2]["w"], r(ls[2]["b"]), sc, of)


def kernel(node_features, edge_features, edge_index, params):
    senders = edge_index[0]
    receivers = edge_index[1]
    sidx_s = senders.reshape(NS, CPS, CHUNK)
    ridx_s = receivers.reshape(NS, CPS, CHUNK)
    sidx_g4 = senders.reshape(NS, N_GRP, GRP_CH, CHUNK)
    ridx_g4 = receivers.reshape(NS, N_GRP, GRP_CH, CHUNK)
    zeros = jnp.zeros((N_PAD, D), jnp.float32)
    one_row = jnp.ones((1, D), jnp.float32)

    nf = jnp.zeros((N_PAD, D), jnp.float32).at[:N_NODES, :27].set(node_features)
    ef = jnp.zeros((N_EDGES, D), jnp.float32).at[:, :4].set(edge_features)

    def enc(p, x, block):
        w0, b0, w1, b1, w2, b2, sc, of = _mlp_args(p)
        w0p = jnp.zeros((D, D), jnp.float32).at[:w0.shape[0]].set(w0)
        return _mlp3(x, w0p, b0, w1, b1, w2, b2, sc.reshape(1, D),
                     of.reshape(1, D), True, block)

    nodes = enc(params["enc_embed_node"], nf, BN)
    edges = enc(params["enc_embed_edge"], ef, BE)

    def gn_step(p, nodes, edges, include_sent):
        sc_gather2, sc_segsum2 = _sc_kernels()
        ew0, eb0, ew1, eb1, ew2, eb2, esc, eof = _mlp_args(p["edge"])
        ps, pr = _project(nodes, ew0[D:2 * D], ew0[2 * D:])
        gs, gr = sc_gather2(ps, pr, sidx_s, ridx_s)
        new_edges = _edge_update(edges, gs, gr, ew0[:D], eb0, ew1, eb1,
                                 ew2, eb2, esc.reshape(1, D), eof.reshape(1, D))
        recv, sent = sc_segsum2(new_edges, ridx_g4, sidx_g4, zeros)
        nw0, nb0, nw1, nb1, nw2, nb2, nsc, nof = _mlp_args(p["node"])
        if include_sent:
            parts = [recv, sent]
            w0s = [nw0[:D], nw0[D:2 * D], nw0[2 * D:]]
        else:
            parts = [recv]
            w0s = [nw0[:D], nw0[D:2 * D]]
        new_nodes = _node_update(nodes, parts, w0s, nb0, nw1, nb1, nw2, nb2,
                                 nsc.reshape(1, D), nof.reshape(1, D))
        return new_nodes, new_edges

    nodes, edges = gn_step(params["enc_gn"], nodes, edges, False)
    for i in range(8):
        nodes, edges = gn_step(params["proc_gn"][i], nodes, edges, True)
    nodes, edges = gn_step(params["dec_gn"], nodes, edges, False)

    dp = params["dec_out"]
    ls = dp["layers"]
    w2p = jnp.zeros((D, D), jnp.float32).at[:, :5].set(ls[2]["w"])
    b2p = jnp.zeros((1, D), jnp.float32).at[0, :5].set(ls[2]["b"])
    out = _mlp3(nodes, ls[0]["w"], ls[0]["b"].reshape(1, D),
                ls[1]["w"], ls[1]["b"].reshape(1, D), w2p, b2p,
                one_row, one_row, False, BN)
    return out[:N_NODES, :5]


# speedup vs baseline: 4.1905x; 1.1056x over previous
import functools

import jax
import jax.numpy as jnp
from jax import lax
from jax.experimental import pallas as pl
from jax.experimental.pallas import tpu as pltpu
from jax.experimental.pallas import tpu_sc as plsc

N_NODES = 10000
N_PAD = 10240
N_EDGES = 320000
E_HALF = N_EDGES // 2
D = 128

NC, NS = 2, 16
CHUNK = 80
NBUF = 5
NBUF_S = 2
ROWS_PER_SUB = N_PAD // NS

BN = 1280
BE = 1280


def _swish(x):
    return x * jax.nn.sigmoid(x)


def _ln(h, scale, off):
    mu = jnp.mean(h, axis=-1, keepdims=True)
    var = jnp.mean((h - mu) ** 2, axis=-1, keepdims=True)
    return (h - mu) / jnp.sqrt(var + 1e-5) * scale + off


def _dot(a, b):
    return jnp.dot(a, b, preferred_element_type=jnp.float32)


def _full(shape):
    return pl.BlockSpec(shape, lambda i: (0, 0))


def _rows(block):
    return pl.BlockSpec((block, D), lambda i: (i, 0))


def _mlp3_kernel(x_ref, w0, b0, w1, b1, w2, b2, sc, of, o_ref, *, use_ln):
    h = _swish(_dot(x_ref[...], w0[...]) + b0[...])
    h = _swish(_dot(h, w1[...]) + b1[...])
    h = _dot(h, w2[...]) + b2[...]
    o_ref[...] = _ln(h, sc[...], of[...]) if use_ln else h


def _mlp3(x, w0, b0, w1, b1, w2, b2, sc, of, use_ln, block):
    n = x.shape[0]
    return pl.pallas_call(
        functools.partial(_mlp3_kernel, use_ln=use_ln),
        out_shape=jax.ShapeDtypeStruct((n, D), jnp.float32),
        grid=(n // block,),
        in_specs=[_rows(block), _full(w0.shape), _full((1, D)),
                  _full(w1.shape), _full((1, D)), _full(w2.shape),
                  _full((1, D)), _full((1, D)), _full((1, D))],
        out_specs=_rows(block),
    )(x, w0, b0, w1, b1, w2, b2, sc, of)


def _edge_update_kernel(e_ref, gs_ref, gr_ref, w0, b0, w1, b1, w2, b2, sc, of,
                        o_ref):
    e = e_ref[...]
    ein = jnp.concatenate([e, gs_ref[...], gr_ref[...]], axis=-1)
    h = _swish(_dot(ein, w0[...]) + b0[...])
    h = _swish(_dot(h, w1[...]) + b1[...])
    h = _dot(h, w2[...]) + b2[...]
    o_ref[...] = e + _ln(h, sc[...], of[...])


def _edge_update(e, gs, gr, w0, b0, w1, b1, w2, b2, sc, of):
    n = e.shape[0]
    return pl.pallas_call(
        _edge_update_kernel,
        out_shape=jax.ShapeDtypeStruct((n, D), jnp.float32),
        grid=(n // BE,),
        in_specs=[_rows(BE), _rows(BE), _rows(BE),
                  _full((3 * D, D)), _full((1, D)), _full((D, D)),
                  _full((1, D)), _full((D, D)), _full((1, D)), _full((1, D)),
                  _full((1, D))],
        out_specs=_rows(BE),
    )(e, gs, gr, w0, b0, w1, b1, w2, b2, sc, of)


def _node_update_kernel(n_ref, parts_refs, w0, b0, w1, b1, w2, b2, sc, of,
                        o_ref, *, nsum):
    n = n_ref[...]
    feats = [n]
    for j in range(nsum):
        f = parts_refs[2 * j][...] + parts_refs[2 * j + 1][...]
        feats.append(f)
    h = _swish(_dot(jnp.concatenate(feats, axis=-1), w0[...]) + b0[...])
    h = _swish(_dot(h, w1[...]) + b1[...])
    h = _dot(h, w2[...]) + b2[...]
    o_ref[...] = n + _ln(h, sc[...], of[...])


def _node_update(n, parts, w0, b0, w1, b1, w2, b2, sc, of):
    k = len(parts)
    nsum = k // 2

    def body(*refs):
        n_ref = refs[0]
        parts_refs = refs[1:1 + k]
        rest = refs[1 + k:]
        _node_update_kernel(n_ref, parts_refs, *rest, nsum=nsum)

    return pl.pallas_call(
        body,
        out_shape=jax.ShapeDtypeStruct((N_PAD, D), jnp.float32),
        grid=(N_PAD // BN,),
        in_specs=[_rows(BN)] + [_rows(BN)] * k +
                 [_full(((1 + nsum) * D, D)),
                  _full((1, D)), _full((D, D)), _full((1, D)),
                  _full((D, D)), _full((1, D)), _full((1, D)), _full((1, D))],
        out_specs=_rows(BN),
    )(n, *parts, w0, b0, w1, b1, w2, b2, sc, of)


@functools.lru_cache(maxsize=None)
def _sc_kernels(n_edges):
    mesh = plsc.VectorSubcoreMesh(core_axis_name="c", subcore_axis_name="s",
                                  num_cores=NC, num_subcores=NS)
    cps = n_edges // CHUNK // NS
    n_outer_s = (cps + NBUF_S - 1) // NBUF_S

    @functools.partial(
        pl.kernel,
        out_type=(jax.ShapeDtypeStruct((n_edges, D), jnp.float32),
                  jax.ShapeDtypeStruct((n_edges, D), jnp.float32)),
        mesh=mesh,
        scratch_types=[
            pltpu.VMEM((cps, CHUNK), jnp.int32),
            pltpu.VMEM((NBUF, CHUNK, D), jnp.float32),
            pltpu.SemaphoreType.DMA((NBUF,)),
            pltpu.SemaphoreType.DMA,
        ],
    )
    def sc_gather2(ps_hbm, pr_hbm, sidx_hbm, ridx_hbm, gs_hbm, gr_hbm,
                   idx_v, buf, sem_g, sem_w):
        c = lax.axis_index("c")
        s = lax.axis_index("s")

        def pipeline(tbl_hbm, idx3_hbm, out_hbm):
            pltpu.sync_copy(idx3_hbm.at[s], idx_v)
            for b in range(NBUF):
                pltpu.async_copy(tbl_hbm.at[idx_v.at[b]], buf.at[b],
                                 sem_g.at[b])

            def outer(g, _):
                for b in range(NBUF):
                    k = g * NBUF + b
                    pltpu.make_async_copy(
                        tbl_hbm.at[idx_v.at[k]], buf.at[b],
                        sem_g.at[b]).wait()
                    base = (s * cps + k) * CHUNK
                    pltpu.async_copy(buf.at[b],
                                     out_hbm.at[pl.ds(base, CHUNK)],
                                     sem_w).wait()
                    nxt = k + NBUF

                    @pl.when(nxt < cps)
                    def _():
                        pltpu.async_copy(tbl_hbm.at[idx_v.at[nxt]],
                                         buf.at[b], sem_g.at[b])
                return ()

            lax.fori_loop(0, cps // NBUF, outer, (), unroll=False)

        @pl.when(c == 0)
        def _():
            pipeline(ps_hbm, sidx_hbm, gs_hbm)

        @pl.when(c == 1)
        def _():
            pipeline(pr_hbm, ridx_hbm, gr_hbm)

    @functools.partial(
        pl.kernel,
        out_type=(jax.ShapeDtypeStruct((N_PAD, D), jnp.float32),
                  jax.ShapeDtypeStruct((N_PAD, D), jnp.float32)),
        mesh=mesh,
        scratch_types=[
            pltpu.VMEM_SHARED((N_PAD, D), jnp.float32),
            pltpu.VMEM((cps, CHUNK), jnp.int32),
            pltpu.VMEM((NBUF_S, CHUNK, D), jnp.float32),
            pltpu.SemaphoreType.DMA((NBUF_S,)),
            pltpu.SemaphoreType.DMA,
        ],
    )
    def sc_segsum2(msgs_hbm, ridx_hbm, sidx_hbm, zeros_hbm, recv_hbm,
                   sent_hbm, acc, idx_v, rows, sem_g, sem_w):
        c = lax.axis_index("c")
        s = lax.axis_index("s")
        r0 = s * ROWS_PER_SUB
        pltpu.sync_copy(zeros_hbm.at[pl.ds(r0, ROWS_PER_SUB)],
                        acc.at[pl.ds(r0, ROWS_PER_SUB)])

        @pl.when(c == 0)
        def _():
            pltpu.sync_copy(ridx_hbm.at[s], idx_v)

        @pl.when(c == 1)
        def _():
            pltpu.sync_copy(sidx_hbm.at[s], idx_v)

        plsc.subcore_barrier()

        e0 = s * cps * CHUNK
        for b in range(NBUF_S):
            pltpu.async_copy(msgs_hbm.at[pl.ds(e0 + b * CHUNK, CHUNK)],
                             rows.at[b], sem_g.at[b])

        def outer(g, _):
            for b in range(NBUF_S):
                k = g * NBUF_S + b

                @pl.when(k < cps)
                def _():
                    pltpu.make_async_copy(
                        msgs_hbm.at[pl.ds(0, CHUNK)], rows.at[b],
                        sem_g.at[b]).wait()
                    pltpu.async_copy(rows.at[b], acc.at[idx_v.at[k]], sem_w,
                                     add=True).wait()
                    nxt = k + NBUF_S

                    @pl.when(nxt < cps)
                    def _():
                        pltpu.async_copy(
                            msgs_hbm.at[pl.ds(e0 + nxt * CHUNK, CHUNK)],
                            rows.at[b], sem_g.at[b])
            return ()

        lax.fori_loop(0, n_outer_s, outer, (), unroll=False)

        plsc.subcore_barrier()

        @pl.when(c == 0)
        def _():
            pltpu.sync_copy(acc.at[pl.ds(r0, ROWS_PER_SUB)],
                            recv_hbm.at[pl.ds(r0, ROWS_PER_SUB)])

        @pl.when(c == 1)
        def _():
            pltpu.sync_copy(acc.at[pl.ds(r0, ROWS_PER_SUB)],
                            sent_hbm.at[pl.ds(r0, ROWS_PER_SUB)])

    return sc_gather2, sc_segsum2


def _mlp_args(p):
    ls = p["layers"]
    sc = p.get("ln_scale")
    of = p.get("ln_offset")
    r = lambda v: v.reshape(1, D) if v is not None else None
    return (ls[0]["w"], r(ls[0]["b"]), ls[1]["w"], r(ls[1]["b"]),
            ls[2]["w"], r(ls[2]["b"]), sc, of)


def kernel(node_features, edge_features, edge_index, params):
    senders = edge_index[0]
    receivers = edge_index[1]
    cps = E_HALF // CHUNK // NS
    sidx = senders.reshape(2, NS, cps, CHUNK)
    ridx = receivers.reshape(2, NS, cps, CHUNK)
    zeros = jnp.zeros((N_PAD, D), jnp.float32)
    one_row = jnp.ones((1, D), jnp.float32)

    nf = jnp.zeros((N_PAD, D), jnp.float32).at[:N_NODES, :27].set(node_features)
    ef = [jnp.zeros((E_HALF, D), jnp.float32)
          .at[:, :4].set(edge_features[h * E_HALF:(h + 1) * E_HALF])
          for h in range(2)]

    def enc(p, x, block):
        w0, b0, w1, b1, w2, b2, sc, of = _mlp_args(p)
        w0p = jnp.zeros((D, D), jnp.float32).at[:w0.shape[0]].set(w0)
        return _mlp3(x, w0p, b0, w1, b1, w2, b2, sc.reshape(1, D),
                     of.reshape(1, D), True, block)

    nodes = enc(params["enc_embed_node"], nf, BN)
    edges = [enc(params["enc_embed_edge"], ef[h], BE) for h in range(2)]

    def gn_step(p, nodes, edges, include_sent):
        sc_gather2, sc_segsum2 = _sc_kernels(E_HALF)
        ew0, eb0, ew1, eb1, ew2, eb2, esc, eof = _mlp_args(p["edge"])

        new_edges, recvs, sents = [], [], []
        g = [sc_gather2(nodes, nodes, sidx[h], ridx[h]) for h in range(2)]
        for h in range(2):
            ne = _edge_update(edges[h], g[h][0], g[h][1], ew0, eb0,
                              ew1, eb1, ew2, eb2, esc.reshape(1, D),
                              eof.reshape(1, D))
            new_edges.append(ne)
            rv, st = sc_segsum2(ne, ridx[h], sidx[h], zeros)
            recvs.append(rv)
            sents.append(st)

        nw0, nb0, nw1, nb1, nw2, nb2, nsc, nof = _mlp_args(p["node"])
        parts = recvs + sents if include_sent else recvs
        new_nodes = _node_update(nodes, parts, nw0, nb0, nw1, nb1, nw2, nb2,
                                 nsc.reshape(1, D), nof.reshape(1, D))
        return new_nodes, new_edges

    nodes, edges = gn_step(params["enc_gn"], nodes, edges, False)
    for i in range(8):
        nodes, edges = gn_step(params["proc_gn"][i], nodes, edges, True)
    nodes, edges = gn_step(params["dec_gn"], nodes, edges, False)

    dp = params["dec_out"]
    ls = dp["layers"]
    w2p = jnp.zeros((D, D), jnp.float32).at[:, :5].set(ls[2]["w"])
    b2p = jnp.zeros((1, D), jnp.float32).at[0, :5].set(ls[2]["b"])
    out = _mlp3(nodes, ls[0]["w"], ls[0]["b"].reshape(1, D),
                ls[1]["w"], ls[1]["b"].reshape(1, D), w2p, b2p,
                one_row, one_row, False, BN)
    return out[:N_NODES, :5]


# speedup vs baseline: 4.1933x; 1.0007x over previous
import functools

import jax
import jax.numpy as jnp
from jax import lax
from jax.experimental import pallas as pl
from jax.experimental.pallas import tpu as pltpu
from jax.experimental.pallas import tpu_sc as plsc

N_NODES = 10000
N_PAD = 10240
N_EDGES = 320000
E_HALF = N_EDGES // 2
D = 128

NC, NS = 2, 16
CHUNK = 80
NBUF_G = 5
NBUF_S = 2
ROWS_PER_SUB = N_PAD // NS

BN = 1280
BE = 1280


def _swish(x):
    return x * jax.nn.sigmoid(x)


def _ln(h, scale, off):
    mu = jnp.mean(h, axis=-1, keepdims=True)
    var = jnp.mean((h - mu) ** 2, axis=-1, keepdims=True)
    return (h - mu) / jnp.sqrt(var + 1e-5) * scale + off


def _dot(a, b):
    return jnp.dot(a, b, preferred_element_type=jnp.float32)


def _full(shape):
    return pl.BlockSpec(shape, lambda i: (0, 0))


def _rows(block):
    return pl.BlockSpec((block, D), lambda i: (i, 0))


def _mlp3_kernel(x_ref, w0, b0, w1, b1, w2, b2, sc, of, o_ref, *, use_ln):
    h = _swish(_dot(x_ref[...], w0[...]) + b0[...])
    h = _swish(_dot(h, w1[...]) + b1[...])
    h = _dot(h, w2[...]) + b2[...]
    o_ref[...] = _ln(h, sc[...], of[...]) if use_ln else h


def _mlp3(x, w0, b0, w1, b1, w2, b2, sc, of, use_ln, block):
    n = x.shape[0]
    return pl.pallas_call(
        functools.partial(_mlp3_kernel, use_ln=use_ln),
        out_shape=jax.ShapeDtypeStruct((n, D), jnp.float32),
        grid=(n // block,),
        in_specs=[_rows(block), _full(w0.shape), _full((1, D)),
                  _full(w1.shape), _full((1, D)), _full(w2.shape),
                  _full((1, D)), _full((1, D)), _full((1, D))],
        out_specs=_rows(block),
    )(x, w0, b0, w1, b1, w2, b2, sc, of)


def _edge_update_kernel(e_ref, gs_ref, gr_ref, w0, b0, w1, b1, w2, b2, sc, of,
                        o_ref):
    e = e_ref[...]
    ein = jnp.concatenate([e, gs_ref[...], gr_ref[...]], axis=-1)
    h = _swish(_dot(ein, w0[...]) + b0[...])
    h = _swish(_dot(h, w1[...]) + b1[...])
    h = _dot(h, w2[...]) + b2[...]
    o_ref[...] = e + _ln(h, sc[...], of[...])


def _edge_update(e, gs, gr, w0, b0, w1, b1, w2, b2, sc, of):
    n = e.shape[0]
    return pl.pallas_call(
        _edge_update_kernel,
        out_shape=jax.ShapeDtypeStruct((n, D), jnp.float32),
        grid=(n // BE,),
        in_specs=[_rows(BE), _rows(BE), _rows(BE),
                  _full((3 * D, D)), _full((1, D)), _full((D, D)),
                  _full((1, D)), _full((D, D)), _full((1, D)), _full((1, D)),
                  _full((1, D))],
        out_specs=_rows(BE),
    )(e, gs, gr, w0, b0, w1, b1, w2, b2, sc, of)


def _node_update_kernel(n_ref, parts_refs, w0, b0, w1, b1, w2, b2, sc, of,
                        o_ref, *, nsum):
    n = n_ref[...]
    feats = [n]
    for j in range(nsum):
        f = parts_refs[2 * j][...] + parts_refs[2 * j + 1][...]
        feats.append(f)
    h = _swish(_dot(jnp.concatenate(feats, axis=-1), w0[...]) + b0[...])
    h = _swish(_dot(h, w1[...]) + b1[...])
    h = _dot(h, w2[...]) + b2[...]
    o_ref[...] = n + _ln(h, sc[...], of[...])


def _node_update(n, parts, w0, b0, w1, b1, w2, b2, sc, of):
    k = len(parts)
    nsum = k // 2

    def body(*refs):
        n_ref = refs[0]
        parts_refs = refs[1:1 + k]
        rest = refs[1 + k:]
        _node_update_kernel(n_ref, parts_refs, *rest, nsum=nsum)

    return pl.pallas_call(
        body,
        out_shape=jax.ShapeDtypeStruct((N_PAD, D), jnp.float32),
        grid=(N_PAD // BN,),
        in_specs=[_rows(BN)] + [_rows(BN)] * k +
                 [_full(((1 + nsum) * D, D)),
                  _full((1, D)), _full((D, D)), _full((1, D)),
                  _full((D, D)), _full((1, D)), _full((1, D)), _full((1, D))],
        out_specs=_rows(BN),
    )(n, *parts, w0, b0, w1, b1, w2, b2, sc, of)


@functools.lru_cache(maxsize=None)
def _sc_kernels(n_edges):
    mesh = plsc.VectorSubcoreMesh(core_axis_name="c", subcore_axis_name="s",
                                  num_cores=NC, num_subcores=NS)
    cps = n_edges // CHUNK // NS
    n_outer_s = (cps + NBUF_S - 1) // NBUF_S

    @functools.partial(
        pl.kernel,
        out_type=(jax.ShapeDtypeStruct((n_edges, D), jnp.float32),
                  jax.ShapeDtypeStruct((n_edges, D), jnp.float32)),
        mesh=mesh,
        scratch_types=[
            pltpu.VMEM((cps, CHUNK), jnp.int32),
            pltpu.VMEM((NBUF_G, CHUNK, D), jnp.float32),
            pltpu.SemaphoreType.DMA((NBUF_G,)),
            pltpu.SemaphoreType.DMA,
        ],
    )
    def sc_gather2(tbl_hbm, sidx_hbm, ridx_hbm, gs_hbm, gr_hbm,
                   idx_v, buf, sem_g, sem_w):
        c = lax.axis_index("c")
        s = lax.axis_index("s")

        def pipeline(idx3_hbm, out_hbm):
            pltpu.sync_copy(idx3_hbm.at[s], idx_v)
            for b in range(NBUF_G):
                pltpu.async_copy(tbl_hbm.at[idx_v.at[b]], buf.at[b],
                                 sem_g.at[b])

            def outer(g, _):
                for b in range(NBUF_G):
                    k = g * NBUF_G + b
                    pltpu.make_async_copy(
                        tbl_hbm.at[idx_v.at[k]], buf.at[b],
                        sem_g.at[b]).wait()
                    base = (s * cps + k) * CHUNK
                    pltpu.async_copy(buf.at[b],
                                     out_hbm.at[pl.ds(base, CHUNK)],
                                     sem_w).wait()
                    nxt = k + NBUF_G

                    @pl.when(nxt < cps)
                    def _():
                        pltpu.async_copy(tbl_hbm.at[idx_v.at[nxt]],
                                         buf.at[b], sem_g.at[b])
                return ()

            lax.fori_loop(0, cps // NBUF_G, outer, (), unroll=False)

        @pl.when(c == 0)
        def _():
            pipeline(sidx_hbm, gs_hbm)

        @pl.when(c == 1)
        def _():
            pipeline(ridx_hbm, gr_hbm)

    @functools.partial(
        pl.kernel,
        out_type=(jax.ShapeDtypeStruct((N_PAD, D), jnp.float32),
                  jax.ShapeDtypeStruct((N_PAD, D), jnp.float32)),
        mesh=mesh,
        scratch_types=[
            pltpu.VMEM_SHARED((N_PAD, D), jnp.float32),
            pltpu.VMEM((cps, CHUNK), jnp.int32),
            pltpu.VMEM((NBUF_S, CHUNK, D), jnp.float32),
            pltpu.SemaphoreType.DMA((NBUF_S,)),
            pltpu.SemaphoreType.DMA,
        ],
    )
    def sc_segsum2(msgs_hbm, ridx_hbm, sidx_hbm, zeros_hbm, recv_hbm,
                   sent_hbm, acc, idx_v, rows, sem_g, sem_w):
        c = lax.axis_index("c")
        s = lax.axis_index("s")
        r0 = s * ROWS_PER_SUB
        pltpu.sync_copy(zeros_hbm.at[pl.ds(r0, ROWS_PER_SUB)],
                        acc.at[pl.ds(r0, ROWS_PER_SUB)])

        @pl.when(c == 0)
        def _():
            pltpu.sync_copy(ridx_hbm.at[s], idx_v)

        @pl.when(c == 1)
        def _():
            pltpu.sync_copy(sidx_hbm.at[s], idx_v)

        plsc.subcore_barrier()

        e0 = s * cps * CHUNK
        for b in range(NBUF_S):
            pltpu.async_copy(msgs_hbm.at[pl.ds(e0 + b * CHUNK, CHUNK)],
                             rows.at[b], sem_g.at[b])

        def outer(g, _):
            for b in range(NBUF_S):
                k = g * NBUF_S + b

                @pl.when(k < cps)
                def _():
                    pltpu.make_async_copy(
                        msgs_hbm.at[pl.ds(0, CHUNK)], rows.at[b],
                        sem_g.at[b]).wait()
                    pltpu.async_copy(rows.at[b], acc.at[idx_v.at[k]], sem_w,
                                     add=True).wait()
                    nxt = k + NBUF_S

                    @pl.when(nxt < cps)
                    def _():
                        pltpu.async_copy(
                            msgs_hbm.at[pl.ds(e0 + nxt * CHUNK, CHUNK)],
                            rows.at[b], sem_g.at[b])
            return ()

        lax.fori_loop(0, n_outer_s, outer, (), unroll=False)

        plsc.subcore_barrier()

        @pl.when(c == 0)
        def _():
            pltpu.sync_copy(acc.at[pl.ds(r0, ROWS_PER_SUB)],
                            recv_hbm.at[pl.ds(r0, ROWS_PER_SUB)])

        @pl.when(c == 1)
        def _():
            pltpu.sync_copy(acc.at[pl.ds(r0, ROWS_PER_SUB)],
                            sent_hbm.at[pl.ds(r0, ROWS_PER_SUB)])

    return sc_gather2, sc_segsum2


def _mlp_args(p):
    ls = p["layers"]
    sc = p.get("ln_scale")
    of = p.get("ln_offset")
    r = lambda v: v.reshape(1, D) if v is not None else None
    return (ls[0]["w"], r(ls[0]["b"]), ls[1]["w"], r(ls[1]["b"]),
            ls[2]["w"], r(ls[2]["b"]), sc, of)


def kernel(node_features, edge_features, edge_index, params):
    senders = edge_index[0]
    receivers = edge_index[1]
    cps = E_HALF // CHUNK // NS
    sidx = senders.reshape(2, NS, cps, CHUNK)
    ridx = receivers.reshape(2, NS, cps, CHUNK)
    zeros = jnp.zeros((N_PAD, D), jnp.float32)
    one_row = jnp.ones((1, D), jnp.float32)

    nf = jnp.zeros((N_PAD, D), jnp.float32).at[:N_NODES, :27].set(node_features)
    ef = [jnp.zeros((E_HALF, D), jnp.float32)
          .at[:, :4].set(edge_features[h * E_HALF:(h + 1) * E_HALF])
          for h in range(2)]

    def enc(p, x, block):
        w0, b0, w1, b1, w2, b2, sc, of = _mlp_args(p)
        w0p = jnp.zeros((D, D), jnp.float32).at[:w0.shape[0]].set(w0)
        return _mlp3(x, w0p, b0, w1, b1, w2, b2, sc.reshape(1, D),
                     of.reshape(1, D), True, block)

    nodes = enc(params["enc_embed_node"], nf, BN)
    edges = [enc(params["enc_embed_edge"], ef[h], BE) for h in range(2)]

    def gn_step(p, nodes, edges, include_sent):
        sc_gather2, sc_segsum2 = _sc_kernels(E_HALF)
        ew0, eb0, ew1, eb1, ew2, eb2, esc, eof = _mlp_args(p["edge"])
        new_edges, recvs, sents = [], [], []
        g = [sc_gather2(nodes, sidx[h], ridx[h]) for h in range(2)]
        for h in range(2):
            ne = _edge_update(edges[h], g[h][0], g[h][1], ew0, eb0,
                              ew1, eb1, ew2, eb2, esc.reshape(1, D),
                              eof.reshape(1, D))
            new_edges.append(ne)
            rv, st = sc_segsum2(ne, ridx[h], sidx[h], zeros)
            recvs.append(rv)
            sents.append(st)

        nw0, nb0, nw1, nb1, nw2, nb2, nsc, nof = _mlp_args(p["node"])
        parts = recvs + sents if include_sent else recvs
        new_nodes = _node_update(nodes, parts, nw0, nb0, nw1, nb1, nw2, nb2,
                                 nsc.reshape(1, D), nof.reshape(1, D))
        return new_nodes, new_edges

    nodes, edges = gn_step(params["enc_gn"], nodes, edges, False)
    for i in range(8):
        nodes, edges = gn_step(params["proc_gn"][i], nodes, edges, True)
    nodes, edges = gn_step(params["dec_gn"], nodes, edges, False)

    dp = params["dec_out"]
    ls = dp["layers"]
    w2p = jnp.zeros((D, D), jnp.float32).at[:, :5].set(ls[2]["w"])
    b2p = jnp.zeros((1, D), jnp.float32).at[0, :5].set(ls[2]["b"])
    out = _mlp3(nodes, ls[0]["w"], ls[0]["b"].reshape(1, D),
                ls[1]["w"], ls[1]["b"].reshape(1, D), w2p, b2p,
                one_row, one_row, False, BN)
    return out[:N_NODES, :5]


# speedup vs baseline: 4.1955x; 1.0005x over previous
import functools

import jax
import jax.numpy as jnp
from jax import lax
from jax.experimental import pallas as pl
from jax.experimental.pallas import tpu as pltpu
from jax.experimental.pallas import tpu_sc as plsc

N_NODES = 10000
N_PAD = 10240
N_EDGES = 320000
E_HALF = N_EDGES // 2
D = 128

NC, NS = 2, 16
CHUNK = 80
NBUF_G = 5
AH = 2
NBUF_S = 2
ROWS_PER_SUB = N_PAD // NS

BN = 1280
BE = 1280


def _swish(x):
    return x * jax.nn.sigmoid(x)


def _ln(h, scale, off):
    mu = jnp.mean(h, axis=-1, keepdims=True)
    var = jnp.mean((h - mu) ** 2, axis=-1, keepdims=True)
    return (h - mu) / jnp.sqrt(var + 1e-5) * scale + off


def _dot(a, b):
    return jnp.dot(a, b, preferred_element_type=jnp.float32)


def _full(shape):
    return pl.BlockSpec(shape, lambda i: (0, 0))


def _rows(block):
    return pl.BlockSpec((block, D), lambda i: (i, 0))


def _mlp3_kernel(x_ref, w0, b0, w1, b1, w2, b2, sc, of, o_ref, *, use_ln):
    h = _swish(_dot(x_ref[...], w0[...]) + b0[...])
    h = _swish(_dot(h, w1[...]) + b1[...])
    h = _dot(h, w2[...]) + b2[...]
    o_ref[...] = _ln(h, sc[...], of[...]) if use_ln else h


def _mlp3(x, w0, b0, w1, b1, w2, b2, sc, of, use_ln, block):
    n = x.shape[0]
    return pl.pallas_call(
        functools.partial(_mlp3_kernel, use_ln=use_ln),
        out_shape=jax.ShapeDtypeStruct((n, D), jnp.float32),
        grid=(n // block,),
        in_specs=[_rows(block), _full(w0.shape), _full((1, D)),
                  _full(w1.shape), _full((1, D)), _full(w2.shape),
                  _full((1, D)), _full((1, D)), _full((1, D))],
        out_specs=_rows(block),
    )(x, w0, b0, w1, b1, w2, b2, sc, of)


def _edge_update_kernel(e_ref, gs_ref, gr_ref, w0, b0, w1, b1, w2, b2, sc, of,
                        o_ref):
    e = e_ref[...]
    ein = jnp.concatenate([e, gs_ref[...], gr_ref[...]], axis=-1)
    h = _swish(_dot(ein, w0[...]) + b0[...])
    h = _swish(_dot(h, w1[...]) + b1[...])
    h = _dot(h, w2[...]) + b2[...]
    o_ref[...] = e + _ln(h, sc[...], of[...])


def _edge_update(e, gs, gr, w0, b0, w1, b1, w2, b2, sc, of):
    n = e.shape[0]
    return pl.pallas_call(
        _edge_update_kernel,
        out_shape=jax.ShapeDtypeStruct((n, D), jnp.float32),
        grid=(n // BE,),
        in_specs=[_rows(BE), _rows(BE), _rows(BE),
                  _full((3 * D, D)), _full((1, D)), _full((D, D)),
                  _full((1, D)), _full((D, D)), _full((1, D)), _full((1, D)),
                  _full((1, D))],
        out_specs=_rows(BE),
    )(e, gs, gr, w0, b0, w1, b1, w2, b2, sc, of)


def _node_update_kernel(n_ref, parts_refs, w0, b0, w1, b1, w2, b2, sc, of,
                        o_ref, *, nsum):
    n = n_ref[...]
    feats = [n]
    for j in range(nsum):
        f = parts_refs[2 * j][...] + parts_refs[2 * j + 1][...]
        feats.append(f)
    h = _swish(_dot(jnp.concatenate(feats, axis=-1), w0[...]) + b0[...])
    h = _swish(_dot(h, w1[...]) + b1[...])
    h = _dot(h, w2[...]) + b2[...]
    o_ref[...] = n + _ln(h, sc[...], of[...])


def _node_update(n, parts, w0, b0, w1, b1, w2, b2, sc, of):
    k = len(parts)
    nsum = k // 2

    def body(*refs):
        n_ref = refs[0]
        parts_refs = refs[1:1 + k]
        rest = refs[1 + k:]
        _node_update_kernel(n_ref, parts_refs, *rest, nsum=nsum)

    return pl.pallas_call(
        body,
        out_shape=jax.ShapeDtypeStruct((N_PAD, D), jnp.float32),
        grid=(N_PAD // BN,),
        in_specs=[_rows(BN)] + [_rows(BN)] * k +
                 [_full(((1 + nsum) * D, D)),
                  _full((1, D)), _full((D, D)), _full((1, D)),
                  _full((D, D)), _full((1, D)), _full((1, D)), _full((1, D))],
        out_specs=_rows(BN),
    )(n, *parts, w0, b0, w1, b1, w2, b2, sc, of)


@functools.lru_cache(maxsize=None)
def _sc_kernels(n_edges):
    mesh = plsc.VectorSubcoreMesh(core_axis_name="c", subcore_axis_name="s",
                                  num_cores=NC, num_subcores=NS)
    cps = n_edges // CHUNK // NS
    n_outer_s = (cps + NBUF_S - 1) // NBUF_S

    @functools.partial(
        pl.kernel,
        out_type=(jax.ShapeDtypeStruct((n_edges, D), jnp.float32),
                  jax.ShapeDtypeStruct((n_edges, D), jnp.float32)),
        mesh=mesh,
        scratch_types=[
            pltpu.VMEM((cps, CHUNK), jnp.int32),
            pltpu.VMEM((NBUF_G, CHUNK, D), jnp.float32),
            pltpu.SemaphoreType.DMA((NBUF_G,)),
            pltpu.SemaphoreType.DMA((NBUF_G,)),
        ],
    )
    def sc_gather2(tbl_hbm, sidx_hbm, ridx_hbm, gs_hbm, gr_hbm,
                   idx_v, buf, sem_g, sem_w):
        c = lax.axis_index("c")
        s = lax.axis_index("s")

        def pipeline(idx3_hbm, out_hbm):
            pltpu.sync_copy(idx3_hbm.at[s], idx_v)
            for b in range(NBUF_G):
                pltpu.async_copy(tbl_hbm.at[idx_v.at[b]], buf.at[b],
                                 sem_g.at[b])

            def outer(g, _):
                for b in range(NBUF_G):
                    i = g * NBUF_G + b
                    b2 = (b - AH) % NBUF_G
                    j2 = i - AH

                    @pl.when((j2 >= 0) & (j2 + NBUF_G < cps))
                    def _():
                        pltpu.make_async_copy(
                            buf.at[b2], out_hbm.at[pl.ds(0, CHUNK)],
                            sem_w.at[b2]).wait()
                        pltpu.async_copy(tbl_hbm.at[idx_v.at[j2 + NBUF_G]],
                                         buf.at[b2], sem_g.at[b2])

                    pltpu.make_async_copy(
                        tbl_hbm.at[idx_v.at[i]], buf.at[b],
                        sem_g.at[b]).wait()
                    base = (s * cps + i) * CHUNK
                    pltpu.async_copy(buf.at[b],
                                     out_hbm.at[pl.ds(base, CHUNK)],
                                     sem_w.at[b])
                return ()

            lax.fori_loop(0, cps // NBUF_G, outer, (), unroll=False)
            for b in range(NBUF_G):
                pltpu.make_async_copy(buf.at[b],
                                      out_hbm.at[pl.ds(0, CHUNK)],
                                      sem_w.at[b]).wait()

        @pl.when(c == 0)
        def _():
            pipeline(sidx_hbm, gs_hbm)

        @pl.when(c == 1)
        def _():
            pipeline(ridx_hbm, gr_hbm)

    @functools.partial(
        pl.kernel,
        out_type=(jax.ShapeDtypeStruct((N_PAD, D), jnp.float32),
                  jax.ShapeDtypeStruct((N_PAD, D), jnp.float32)),
        mesh=mesh,
        scratch_types=[
            pltpu.VMEM_SHARED((N_PAD, D), jnp.float32),
            pltpu.VMEM((cps, CHUNK), jnp.int32),
            pltpu.VMEM((NBUF_S, CHUNK, D), jnp.float32),
            pltpu.SemaphoreType.DMA((NBUF_S,)),
            pltpu.SemaphoreType.DMA,
        ],
    )
    def sc_segsum2(msgs_hbm, ridx_hbm, sidx_hbm, zeros_hbm, recv_hbm,
                   sent_hbm, acc, idx_v, rows, sem_g, sem_w):
        c = lax.axis_index("c")
        s = lax.axis_index("s")
        r0 = s * ROWS_PER_SUB
        pltpu.sync_copy(zeros_hbm.at[pl.ds(r0, ROWS_PER_SUB)],
                        acc.at[pl.ds(r0, ROWS_PER_SUB)])

        @pl.when(c == 0)
        def _():
            pltpu.sync_copy(ridx_hbm.at[s], idx_v)

        @pl.when(c == 1)
        def _():
            pltpu.sync_copy(sidx_hbm.at[s], idx_v)

        plsc.subcore_barrier()

        e0 = s * cps * CHUNK
        for b in range(NBUF_S):
            pltpu.async_copy(msgs_hbm.at[pl.ds(e0 + b * CHUNK, CHUNK)],
                             rows.at[b], sem_g.at[b])

        def outer(g, _):
            for b in range(NBUF_S):
                k = g * NBUF_S + b

                @pl.when(k < cps)
                def _():
                    pltpu.make_async_copy(
                        msgs_hbm.at[pl.ds(0, CHUNK)], rows.at[b],
                        sem_g.at[b]).wait()
                    pltpu.async_copy(rows.at[b], acc.at[idx_v.at[k]], sem_w,
                                     add=True).wait()
                    nxt = k + NBUF_S

                    @pl.when(nxt < cps)
                    def _():
                        pltpu.async_copy(
                            msgs_hbm.at[pl.ds(e0 + nxt * CHUNK, CHUNK)],
                            rows.at[b], sem_g.at[b])
            return ()

        lax.fori_loop(0, n_outer_s, outer, (), unroll=False)

        plsc.subcore_barrier()

        @pl.when(c == 0)
        def _():
            pltpu.sync_copy(acc.at[pl.ds(r0, ROWS_PER_SUB)],
                            recv_hbm.at[pl.ds(r0, ROWS_PER_SUB)])

        @pl.when(c == 1)
        def _():
            pltpu.sync_copy(acc.at[pl.ds(r0, ROWS_PER_SUB)],
                            sent_hbm.at[pl.ds(r0, ROWS_PER_SUB)])

    return sc_gather2, sc_segsum2


def _mlp_args(p):
    ls = p["layers"]
    sc = p.get("ln_scale")
    of = p.get("ln_offset")
    r = lambda v: v.reshape(1, D) if v is not None else None
    return (ls[0]["w"], r(ls[0]["b"]), ls[1]["w"], r(ls[1]["b"]),
            ls[2]["w"], r(ls[2]["b"]), sc, of)


def kernel(node_features, edge_features, edge_index, params):
    senders = edge_index[0]
    receivers = edge_index[1]
    cps = E_HALF // CHUNK // NS
    sidx = senders.reshape(2, NS, cps, CHUNK)
    ridx = receivers.reshape(2, NS, cps, CHUNK)
    zeros = jnp.zeros((N_PAD, D), jnp.float32)
    one_row = jnp.ones((1, D), jnp.float32)

    nf = jnp.zeros((N_PAD, D), jnp.float32).at[:N_NODES, :27].set(node_features)
    ef = [jnp.zeros((E_HALF, D), jnp.float32)
          .at[:, :4].set(edge_features[h * E_HALF:(h + 1) * E_HALF])
          for h in range(2)]

    def enc(p, x, block):
        w0, b0, w1, b1, w2, b2, sc, of = _mlp_args(p)
        w0p = jnp.zeros((D, D), jnp.float32).at[:w0.shape[0]].set(w0)
        return _mlp3(x, w0p, b0, w1, b1, w2, b2, sc.reshape(1, D),
                     of.reshape(1, D), True, block)

    nodes = enc(params["enc_embed_node"], nf, BN)
    edges = [enc(params["enc_embed_edge"], ef[h], BE) for h in range(2)]

    def gn_step(p, nodes, edges, include_sent):
        sc_gather2, sc_segsum2 = _sc_kernels(E_HALF)
        ew0, eb0, ew1, eb1, ew2, eb2, esc, eof = _mlp_args(p["edge"])
        new_edges, recvs, sents = [], [], []
        g = [sc_gather2(nodes, sidx[h], ridx[h]) for h in range(2)]
        for h in range(2):
            ne = _edge_update(edges[h], g[h][0], g[h][1], ew0, eb0,
                              ew1, eb1, ew2, eb2, esc.reshape(1, D),
                              eof.reshape(1, D))
            new_edges.append(ne)
            rv, st = sc_segsum2(ne, ridx[h], sidx[h], zeros)
            recvs.append(rv)
            sents.append(st)

        nw0, nb0, nw1, nb1, nw2, nb2, nsc, nof = _mlp_args(p["node"])
        parts = recvs + sents if include_sent else recvs
        new_nodes = _node_update(nodes, parts, nw0, nb0, nw1, nb1, nw2, nb2,
                                 nsc.reshape(1, D), nof.reshape(1, D))
        return new_nodes, new_edges

    nodes, edges = gn_step(params["enc_gn"], nodes, edges, False)
    for i in range(8):
        nodes, edges = gn_step(params["proc_gn"][i], nodes, edges, True)
    nodes, edges = gn_step(params["dec_gn"], nodes, edges, False)

    dp = params["dec_out"]
    ls = dp["layers"]
    w2p = jnp.zeros((D, D), jnp.float32).at[:, :5].set(ls[2]["w"])
    b2p = jnp.zeros((1, D), jnp.float32).at[0, :5].set(ls[2]["b"])
    out = _mlp3(nodes, ls[0]["w"], ls[0]["b"].reshape(1, D),
                ls[1]["w"], ls[1]["b"].reshape(1, D), w2p, b2p,
                one_row, one_row, False, BN)
    return out[:N_NODES, :5]


# speedup vs baseline: 4.2128x; 1.0041x over previous
import functools

import jax
import jax.numpy as jnp
from jax import lax
from jax.experimental import pallas as pl
from jax.experimental.pallas import tpu as pltpu
from jax.experimental.pallas import tpu_sc as plsc

N_NODES = 10000
N_PAD = 10240
N_EDGES = 320000
E_HALF = N_EDGES // 2
D = 128

NC, NS = 2, 16
CHUNK = 80
NBUF_G = 5
AH = 2
NBUF_S = 2
ROWS_PER_SUB = N_PAD // NS

BN = 1280
BE = 1280


def _swish(x):
    return x * jax.nn.sigmoid(x)


def _ln(h, scale, off):
    mu = jnp.mean(h, axis=-1, keepdims=True)
    var = jnp.mean((h - mu) ** 2, axis=-1, keepdims=True)
    return (h - mu) / jnp.sqrt(var + 1e-5) * scale + off


def _dot(a, b):
    return jnp.dot(a, b, preferred_element_type=jnp.float32)


def _full(shape):
    return pl.BlockSpec(shape, lambda i: (0, 0))


def _rows(block):
    return pl.BlockSpec((block, D), lambda i: (i, 0))


def _mlp3_kernel(x_ref, w0, b0, w1, b1, w2, b2, sc, of, o_ref, *, use_ln):
    h = _swish(_dot(x_ref[...], w0[...]) + b0[...])
    h = _swish(_dot(h, w1[...]) + b1[...])
    h = _dot(h, w2[...]) + b2[...]
    o_ref[...] = _ln(h, sc[...], of[...]) if use_ln else h


def _mlp3(x, w0, b0, w1, b1, w2, b2, sc, of, use_ln, block):
    n = x.shape[0]
    return pl.pallas_call(
        functools.partial(_mlp3_kernel, use_ln=use_ln),
        out_shape=jax.ShapeDtypeStruct((n, D), jnp.float32),
        grid=(n // block,),
        in_specs=[_rows(block), _full(w0.shape), _full((1, D)),
                  _full(w1.shape), _full((1, D)), _full(w2.shape),
                  _full((1, D)), _full((1, D)), _full((1, D))],
        out_specs=_rows(block),
    )(x, w0, b0, w1, b1, w2, b2, sc, of)


def _edge_update_kernel(e_ref, gs_ref, gr_ref, w0, b0, w1, b1, w2, b2, sc, of,
                        o_ref):
    e = e_ref[...]
    ein = jnp.concatenate([e, gs_ref[...], gr_ref[...]], axis=-1)
    h = _swish(_dot(ein, w0[...]) + b0[...])
    h = _swish(_dot(h, w1[...]) + b1[...])
    h = _dot(h, w2[...]) + b2[...]
    o_ref[...] = e + _ln(h, sc[...], of[...])


def _edge_update(e, gs, gr, w0, b0, w1, b1, w2, b2, sc, of):
    n = e.shape[0]
    return pl.pallas_call(
        _edge_update_kernel,
        out_shape=jax.ShapeDtypeStruct((n, D), jnp.float32),
        grid=(n // BE,),
        in_specs=[_rows(BE), _rows(BE), _rows(BE),
                  _full((3 * D, D)), _full((1, D)), _full((D, D)),
                  _full((1, D)), _full((D, D)), _full((1, D)), _full((1, D)),
                  _full((1, D))],
        out_specs=_rows(BE),
    )(e, gs, gr, w0, b0, w1, b1, w2, b2, sc, of)


def _node_update_kernel(n_ref, parts_refs, w0, b0, w1, b1, w2, b2, sc, of,
                        o_ref, *, nsum):
    n = n_ref[...]
    feats = [n]
    for j in range(nsum):
        f = parts_refs[2 * j][...] + parts_refs[2 * j + 1][...]
        feats.append(f)
    h = _swish(_dot(jnp.concatenate(feats, axis=-1), w0[...]) + b0[...])
    h = _swish(_dot(h, w1[...]) + b1[...])
    h = _dot(h, w2[...]) + b2[...]
    o_ref[...] = n + _ln(h, sc[...], of[...])


def _node_update(n, parts, w0, b0, w1, b1, w2, b2, sc, of):
    k = len(parts)
    nsum = k // 2

    def body(*refs):
        n_ref = refs[0]
        parts_refs = refs[1:1 + k]
        rest = refs[1 + k:]
        _node_update_kernel(n_ref, parts_refs, *rest, nsum=nsum)

    return pl.pallas_call(
        body,
        out_shape=jax.ShapeDtypeStruct((N_PAD, D), jnp.float32),
        grid=(N_PAD // BN,),
        in_specs=[_rows(BN)] + [_rows(BN)] * k +
                 [_full(((1 + nsum) * D, D)),
                  _full((1, D)), _full((D, D)), _full((1, D)),
                  _full((D, D)), _full((1, D)), _full((1, D)), _full((1, D))],
        out_specs=_rows(BN),
    )(n, *parts, w0, b0, w1, b1, w2, b2, sc, of)


@functools.lru_cache(maxsize=None)
def _sc_kernels(n_edges):
    mesh = plsc.VectorSubcoreMesh(core_axis_name="c", subcore_axis_name="s",
                                  num_cores=NC, num_subcores=NS)
    cps = n_edges // CHUNK // NS
    n_outer_s = (cps + NBUF_S - 1) // NBUF_S

    @functools.partial(
        pl.kernel,
        out_type=(jax.ShapeDtypeStruct((n_edges, D), jnp.float32),
                  jax.ShapeDtypeStruct((n_edges, D), jnp.float32)),
        mesh=mesh,
        scratch_types=[
            pltpu.VMEM((cps, CHUNK), jnp.int32),
            pltpu.VMEM((NBUF_G, CHUNK, D), jnp.float32),
            pltpu.SemaphoreType.DMA((NBUF_G,)),
            pltpu.SemaphoreType.DMA((NBUF_G,)),
        ],
    )
    def sc_gather2(tbl_hbm, sidx_hbm, ridx_hbm, gs_hbm, gr_hbm,
                   idx_v, buf, sem_g, sem_w):
        c = lax.axis_index("c")
        s = lax.axis_index("s")

        def pipeline(idx3_hbm, out_hbm):
            pltpu.sync_copy(idx3_hbm.at[s], idx_v)
            for b in range(NBUF_G):
                pltpu.async_copy(tbl_hbm.at[idx_v.at[b]], buf.at[b],
                                 sem_g.at[b])

            def outer(g, _):
                for b in range(NBUF_G):
                    i = g * NBUF_G + b
                    b2 = (b - AH) % NBUF_G
                    j2 = i - AH

                    @pl.when((j2 >= 0) & (j2 + NBUF_G < cps))
                    def _():
                        pltpu.make_async_copy(
                            buf.at[b2], out_hbm.at[pl.ds(0, CHUNK)],
                            sem_w.at[b2]).wait()
                        pltpu.async_copy(tbl_hbm.at[idx_v.at[j2 + NBUF_G]],
                                         buf.at[b2], sem_g.at[b2])

                    pltpu.make_async_copy(
                        tbl_hbm.at[idx_v.at[i]], buf.at[b],
                        sem_g.at[b]).wait()
                    base = (s * cps + i) * CHUNK
                    pltpu.async_copy(buf.at[b],
                                     out_hbm.at[pl.ds(base, CHUNK)],
                                     sem_w.at[b])
                return ()

            lax.fori_loop(0, cps // NBUF_G, outer, (), unroll=False)
            for b in range(NBUF_G):
                pltpu.make_async_copy(buf.at[b],
                                      out_hbm.at[pl.ds(0, CHUNK)],
                                      sem_w.at[b]).wait()

        @pl.when(c == 0)
        def _():
            pipeline(sidx_hbm, gs_hbm)

        @pl.when(c == 1)
        def _():
            pipeline(ridx_hbm, gr_hbm)

    @functools.partial(
        pl.kernel,
        out_type=(jax.ShapeDtypeStruct((N_PAD, D), jnp.float32),
                  jax.ShapeDtypeStruct((N_PAD, D), jnp.float32)),
        mesh=mesh,
        scratch_types=[
            pltpu.VMEM_SHARED((N_PAD, D), jnp.float32),
            pltpu.VMEM((cps, CHUNK), jnp.int32),
            pltpu.VMEM((NBUF_S, CHUNK, D), jnp.float32),
            pltpu.SemaphoreType.DMA((NBUF_S,)),
            pltpu.SemaphoreType.DMA,
        ],
    )
    def sc_segsum2(msgs_hbm, ridx_hbm, sidx_hbm, zeros_hbm, recv_hbm,
                   sent_hbm, acc, idx_v, rows, sem_g, sem_w):
        c = lax.axis_index("c")
        s = lax.axis_index("s")
        r0 = s * ROWS_PER_SUB
        pltpu.sync_copy(zeros_hbm.at[pl.ds(r0, ROWS_PER_SUB)],
                        acc.at[pl.ds(r0, ROWS_PER_SUB)])

        @pl.when(c == 0)
        def _():
            pltpu.sync_copy(ridx_hbm.at[s], idx_v)

        @pl.when(c == 1)
        def _():
            pltpu.sync_copy(sidx_hbm.at[s], idx_v)

        plsc.subcore_barrier()

        e0 = s * cps * CHUNK
        for b in range(NBUF_S):
            pltpu.async_copy(msgs_hbm.at[pl.ds(e0 + b * CHUNK, CHUNK)],
                             rows.at[b], sem_g.at[b])

        def outer(g, _):
            for b in range(NBUF_S):
                k = g * NBUF_S + b

                @pl.when(k < cps)
                def _():
                    pltpu.make_async_copy(
                        msgs_hbm.at[pl.ds(0, CHUNK)], rows.at[b],
                        sem_g.at[b]).wait()
                    pltpu.async_copy(rows.at[b], acc.at[idx_v.at[k]], sem_w,
                                     add=True).wait()
                    nxt = k + NBUF_S

                    @pl.when(nxt < cps)
                    def _():
                        pltpu.async_copy(
                            msgs_hbm.at[pl.ds(e0 + nxt * CHUNK, CHUNK)],
                            rows.at[b], sem_g.at[b])
            return ()

        lax.fori_loop(0, n_outer_s, outer, (), unroll=False)

        plsc.subcore_barrier()

        @pl.when(c == 0)
        def _():
            pltpu.sync_copy(acc.at[pl.ds(r0, ROWS_PER_SUB)],
                            recv_hbm.at[pl.ds(r0, ROWS_PER_SUB)])

        @pl.when(c == 1)
        def _():
            pltpu.sync_copy(acc.at[pl.ds(r0, ROWS_PER_SUB)],
                            sent_hbm.at[pl.ds(r0, ROWS_PER_SUB)])

    @functools.partial(
        pl.kernel,
        out_type=(jax.ShapeDtypeStruct((N_PAD, D), jnp.float32),
                  jax.ShapeDtypeStruct((N_PAD, D), jnp.float32)),
        mesh=mesh,
        scratch_types=[
            pltpu.VMEM_SHARED((N_PAD, D), jnp.float32),
            pltpu.VMEM((cps, CHUNK), jnp.int32),
            pltpu.VMEM((NBUF_S, CHUNK, D), jnp.float32),
            pltpu.SemaphoreType.DMA((NBUF_S,)),
            pltpu.SemaphoreType.DMA,
        ],
    )
    def sc_segsum_recv(msgs0_hbm, msgs1_hbm, ridx_hbm, zeros_hbm, r0_out,
                       r1_out, acc, idx_v, rows, sem_g, sem_w):
        c = lax.axis_index("c")
        s = lax.axis_index("s")
        w = s * NC + c
        r0 = s * ROWS_PER_SUB
        pltpu.sync_copy(zeros_hbm.at[pl.ds(r0, ROWS_PER_SUB)],
                        acc.at[pl.ds(r0, ROWS_PER_SUB)])
        pltpu.sync_copy(ridx_hbm.at[w], idx_v)
        plsc.subcore_barrier()

        def run(msgs_hbm, wloc):
            e0 = wloc * cps * CHUNK
            for b in range(NBUF_S):
                pltpu.async_copy(msgs_hbm.at[pl.ds(e0 + b * CHUNK, CHUNK)],
                                 rows.at[b], sem_g.at[b])

            def outer(g, _):
                for b in range(NBUF_S):
                    k = g * NBUF_S + b

                    @pl.when(k < cps)
                    def _():
                        pltpu.make_async_copy(
                            msgs_hbm.at[pl.ds(0, CHUNK)], rows.at[b],
                            sem_g.at[b]).wait()
                        pltpu.async_copy(rows.at[b], acc.at[idx_v.at[k]],
                                         sem_w, add=True).wait()
                        nxt = k + NBUF_S

                        @pl.when(nxt < cps)
                        def _():
                            pltpu.async_copy(
                                msgs_hbm.at[pl.ds(e0 + nxt * CHUNK, CHUNK)],
                                rows.at[b], sem_g.at[b])
                return ()

            lax.fori_loop(0, n_outer_s, outer, (), unroll=False)

        @pl.when(w < NS)
        def _():
            run(msgs0_hbm, w)

        @pl.when(w >= NS)
        def _():
            run(msgs1_hbm, w - NS)

        plsc.subcore_barrier()

        @pl.when(c == 0)
        def _():
            pltpu.sync_copy(acc.at[pl.ds(r0, ROWS_PER_SUB)],
                            r0_out.at[pl.ds(r0, ROWS_PER_SUB)])

        @pl.when(c == 1)
        def _():
            pltpu.sync_copy(acc.at[pl.ds(r0, ROWS_PER_SUB)],
                            r1_out.at[pl.ds(r0, ROWS_PER_SUB)])

    return sc_gather2, sc_segsum2, sc_segsum_recv


def _mlp_args(p):
    ls = p["layers"]
    sc = p.get("ln_scale")
    of = p.get("ln_offset")
    r = lambda v: v.reshape(1, D) if v is not None else None
    return (ls[0]["w"], r(ls[0]["b"]), ls[1]["w"], r(ls[1]["b"]),
            ls[2]["w"], r(ls[2]["b"]), sc, of)


def kernel(node_features, edge_features, edge_index, params):
    senders = edge_index[0]
    receivers = edge_index[1]
    cps = E_HALF // CHUNK // NS
    sidx = senders.reshape(2, NS, cps, CHUNK)
    ridx = receivers.reshape(2, NS, cps, CHUNK)
    zeros = jnp.zeros((N_PAD, D), jnp.float32)
    one_row = jnp.ones((1, D), jnp.float32)

    nf = jnp.zeros((N_PAD, D), jnp.float32).at[:N_NODES, :27].set(node_features)
    ef = [jnp.zeros((E_HALF, D), jnp.float32)
          .at[:, :4].set(edge_features[h * E_HALF:(h + 1) * E_HALF])
          for h in range(2)]

    def enc(p, x, block):
        w0, b0, w1, b1, w2, b2, sc, of = _mlp_args(p)
        w0p = jnp.zeros((D, D), jnp.float32).at[:w0.shape[0]].set(w0)
        return _mlp3(x, w0p, b0, w1, b1, w2, b2, sc.reshape(1, D),
                     of.reshape(1, D), True, block)

    nodes = enc(params["enc_embed_node"], nf, BN)
    edges = [enc(params["enc_embed_edge"], ef[h], BE) for h in range(2)]

    ridx_w = receivers.reshape(NC * NS, N_EDGES // CHUNK // (NC * NS), CHUNK)

    def gn_step(p, nodes, edges, include_sent):
        sc_gather2, sc_segsum2, sc_segsum_recv = _sc_kernels(E_HALF)
        ew0, eb0, ew1, eb1, ew2, eb2, esc, eof = _mlp_args(p["edge"])
        new_edges, recvs, sents = [], [], []
        g = [sc_gather2(nodes, sidx[h], ridx[h]) for h in range(2)]
        for h in range(2):
            ne = _edge_update(edges[h], g[h][0], g[h][1], ew0, eb0,
                              ew1, eb1, ew2, eb2, esc.reshape(1, D),
                              eof.reshape(1, D))
            new_edges.append(ne)
            if include_sent:
                rv, st = sc_segsum2(ne, ridx[h], sidx[h], zeros)
                recvs.append(rv)
                sents.append(st)
        if not include_sent:
            recvs = list(sc_segsum_recv(new_edges[0], new_edges[1],
                                        ridx_w, zeros))

        nw0, nb0, nw1, nb1, nw2, nb2, nsc, nof = _mlp_args(p["node"])
        parts = recvs + sents if include_sent else recvs
        new_nodes = _node_update(nodes, parts, nw0, nb0, nw1, nb1, nw2, nb2,
                                 nsc.reshape(1, D), nof.reshape(1, D))
        return new_nodes, new_edges

    nodes, edges = gn_step(params["enc_gn"], nodes, edges, False)
    for i in range(8):
        nodes, edges = gn_step(params["proc_gn"][i], nodes, edges, True)
    nodes, edges = gn_step(params["dec_gn"], nodes, edges, False)

    dp = params["dec_out"]
    ls = dp["layers"]
    w2p = jnp.zeros((D, D), jnp.float32).at[:, :5].set(ls[2]["w"])
    b2p = jnp.zeros((1, D), jnp.float32).at[0, :5].set(ls[2]["b"])
    out = _mlp3(nodes, ls[0]["w"], ls[0]["b"].reshape(1, D),
                ls[1]["w"], ls[1]["b"].reshape(1, D), w2p, b2p,
                one_row, one_row, False, BN)
    return out[:N_NODES, :5]
